# Initial kernel scaffold; baseline (speedup 1.0000x reference)
import jax
import jax.numpy as jnp
from jax.experimental import pallas as pl


def kernel(X, E_real, E_img, R_real, R_img, T_emb):
    raise NotImplementedError("write your pallas kernel here")


# speedup vs baseline: 1.8625x; 1.8625x over previous
import dataclasses
import functools

import jax
import jax.numpy as jnp
import numpy as np
from jax import lax
from jax.experimental import pallas as pl
from jax.experimental.pallas import tpu as pltpu
from jax.experimental.pallas import tpu_sc as plsc

NC = 2
NS = 16
NW = NC * NS
L = 16
CH = 128


def _trig_body(t_ref, cos_ref, sin_ref):
    x = t_ref[...]
    cos_ref[...] = jnp.cos(x)
    sin_ref[...] = jnp.sin(x)


def kernel(X, E_real, E_img, R_real, R_img, T_emb):
    B = X.shape[0]
    dim = E_real.shape[1]
    h_idx = X[:, 0]
    t_idx = X[:, 1]
    r_idx = X[:, 2]
    d_idx = X[:, 3]

    cosT, sinT = pl.pallas_call(
        _trig_body,
        out_shape=[jax.ShapeDtypeStruct(T_emb.shape, jnp.float32)] * 2,
    )(T_emb)

    b_per_w = B // NW
    mesh = plsc.VectorSubcoreMesh(core_axis_name="c", subcore_axis_name="s")

    cp = pltpu.CompilerParams()
    fields = pltpu.CompilerParams.__dataclass_fields__
    if "needs_layout_passes" in fields:
        cp = dataclasses.replace(cp, needs_layout_passes=False)
    if "use_tc_tiling_on_sc" in fields:
        cp = dataclasses.replace(cp, use_tc_tiling_on_sc=False)

    @functools.partial(
        pl.kernel,
        out_type=jax.ShapeDtypeStruct((B,), jnp.float32),
        mesh=mesh,
        compiler_params=cp,
        scratch_types=[
            pltpu.VMEM((b_per_w,), jnp.int32),
            pltpu.VMEM((b_per_w,), jnp.int32),
            pltpu.VMEM((b_per_w,), jnp.int32),
            pltpu.VMEM((b_per_w,), jnp.int32),
            pltpu.VMEM((CH, dim), jnp.float32),
            pltpu.VMEM((CH, dim), jnp.float32),
            pltpu.VMEM((CH, dim), jnp.float32),
            pltpu.VMEM((CH, dim), jnp.float32),
            pltpu.VMEM((CH, dim), jnp.float32),
            pltpu.VMEM((CH, dim), jnp.float32),
            pltpu.VMEM((CH, dim), jnp.float32),
            pltpu.VMEM((CH, dim), jnp.float32),
            pltpu.VMEM((b_per_w,), jnp.float32),
            pltpu.SemaphoreType.DMA,
        ],
    )
    def sc_score(h_hbm, t_hbm, r_hbm, d_hbm, er_hbm, ei_hbm, rr_hbm, ri_hbm,
                 ct_hbm, st_hbm, out_hbm,
                 hi_v, ti_v, ri_v, di_v,
                 hre_v, him_v, tre_v, tim_v, rre_v, rim_v, cos_v, sin_v,
                 out_v, sem):
        iota = lax.iota(jnp.int32, L)
        wid = lax.axis_index("s") * NC + lax.axis_index("c")
        base = wid * b_per_w
        pltpu.sync_copy(h_hbm.at[pl.ds(base, b_per_w)], hi_v)
        pltpu.sync_copy(t_hbm.at[pl.ds(base, b_per_w)], ti_v)
        pltpu.sync_copy(r_hbm.at[pl.ds(base, b_per_w)], ri_v)
        pltpu.sync_copy(d_hbm.at[pl.ds(base, b_per_w)], di_v)

        @pl.loop(0, b_per_w // CH)
        def _chunk(ci):
            off = ci * CH
            copies = [
                pltpu.async_copy(er_hbm.at[hi_v.at[pl.ds(off, CH)]], hre_v, sem),
                pltpu.async_copy(ei_hbm.at[hi_v.at[pl.ds(off, CH)]], him_v, sem),
                pltpu.async_copy(er_hbm.at[ti_v.at[pl.ds(off, CH)]], tre_v, sem),
                pltpu.async_copy(ei_hbm.at[ti_v.at[pl.ds(off, CH)]], tim_v, sem),
                pltpu.async_copy(rr_hbm.at[ri_v.at[pl.ds(off, CH)]], rre_v, sem),
                pltpu.async_copy(ri_hbm.at[ri_v.at[pl.ds(off, CH)]], rim_v, sem),
                pltpu.async_copy(ct_hbm.at[di_v.at[pl.ds(off, CH)]], cos_v, sem),
                pltpu.async_copy(st_hbm.at[di_v.at[pl.ds(off, CH)]], sin_v, sem),
            ]
            for c in copies:
                c.wait()

            @pl.loop(0, CH // L)
            def _group(g):
                acc = jnp.zeros((L,), jnp.float32)
                for l in range(L):
                    w = g * L + l
                    s = jnp.zeros((L,), jnp.float32)
                    for half in range(dim // L):
                        sl = pl.ds(half * L, L)
                        hre = hre_v[w, sl]
                        him = him_v[w, sl]
                        tre = tre_v[w, sl]
                        tim = tim_v[w, sl]
                        c = cos_v[w, sl]
                        sn = sin_v[w, sl]
                        real = (hre - tre) * c - (him - tim) * sn + rre_v[w, sl]
                        imag = (hre + tre) * sn + (him + tim) * c + rim_v[w, sl]
                        s = s + jnp.abs(real) + jnp.abs(imag)
                    tot = jnp.sum(s)
                    acc = jnp.where(iota == l, tot, acc)
                out_v[pl.ds(off + g * L, L)] = acc

        pltpu.sync_copy(out_v, out_hbm.at[pl.ds(base, b_per_w)])

    return sc_score(h_idx, t_idx, r_idx, d_idx, E_real, E_img,
                    R_real, R_img, cosT, sinT)
